# baseline (speedup 1.0000x reference)
import functools

import jax
import jax.numpy as jnp
from jax.experimental import pallas as pl
from jax.experimental.pallas import tpu as pltpu

_FACTOR = 2
_SD = 8
_WP = 128
_CH = 512


def _step_kernel(x_ref, w_ref, b_ref, out_ref, xs_ref, xa_ref, xb_ref,
                 xc_ref, *, W, T, L, F2, SD, FILT):
    C = x_ref.shape[1]
    H = x_ref.shape[2]
    HWp = H * _WP
    GP = _WP
    C3 = 3 * F2
    CH = _CH if (HWp % _CH == 0 and HWp >= _CH) else HWp
    NCH = HWp // CH
    F = FILT

    zc1 = jnp.zeros((C, H, 1), jnp.float32)
    zc2 = jnp.zeros((C, H, _WP - W - 1), jnp.float32)
    for t in range(T):
        xp = jnp.concatenate([zc1, x_ref[t], zc2], axis=2)
        xs_ref[t] = xp.reshape(C, HWp).astype(jnp.bfloat16)
    xa_ref[...] = jnp.zeros(xa_ref.shape, jnp.bfloat16)
    xb_ref[...] = jnp.zeros(xb_ref.shape, jnp.bfloat16)
    xc_ref[...] = jnp.zeros(xc_ref.shape, jnp.bfloat16)

    colv = jax.lax.broadcasted_iota(jnp.int32, (1, CH), 1) % _WP
    real = (colv >= 1) & (colv <= W)

    BLOCKS = ((0, _WP), (F, 0), (2 * F, -_WP))

    def put3(dst_ref, y16, base):
        for blk, off in BLOCKS:
            dst_ref[blk:blk + F, GP + base + off:GP + base + off + CH] = y16

    def conv_chunk(layer, src_ref, base, relu):
        z = jnp.dot(w_ref[layer], src_ref[:, GP + base:GP + base + CH],
                    preferred_element_type=jnp.float32)
        zl = pltpu.roll(z[:F], 1, axis=1)
        zr = pltpu.roll(z[2 * F:], CH - 1, axis=1)
        y = z[F:2 * F] + b_ref[layer][:, :1] + zl + zr
        if relu:
            y = jnp.maximum(y, 0.0)
        return jnp.where(real, y, 0.0)

    def step(t, carry):
        tp = jnp.maximum(t - 1, 0)
        tn = jnp.minimum(t + 1, T - 1)
        fr = jnp.concatenate([xs_ref[tp], xs_ref[t], xs_ref[tn]], axis=0)
        for blk, off in BLOCKS:
            xa_ref[blk + C3 + SD:blk + C3 + SD + 3 * C,
                   GP + off:GP + off + HWp] = fr

        for j in range(NCH):
            base = j * CH
            y = conv_chunk(0, xa_ref, base, relu=True)
            put3(xb_ref, y.astype(jnp.bfloat16), base)
        for j in range(NCH):
            base = j * CH
            y = conv_chunk(1, xb_ref, base, relu=True)
            put3(xc_ref, y.astype(jnp.bfloat16), base)
        for j in range(NCH):
            base = j * CH
            y = conv_chunk(2, xc_ref, base, relu=False)
            fch = xs_ref[t, :, base:base + CH].astype(jnp.float32)
            rgb16 = (y[:C3]
                     + jnp.concatenate([fch] * F2, axis=0)
                     ).astype(jnp.bfloat16)
            st16 = jnp.maximum(y[C3:C3 + SD], 0.0).astype(jnp.bfloat16)
            out_ref[t, :, base:base + CH] = rgb16
            for blk, off in BLOCKS:
                lo = GP + base + off
                xa_ref[blk:blk + C3, lo:lo + CH] = rgb16
                xa_ref[blk + C3:blk + C3 + SD, lo:lo + CH] = st16
        return carry

    jax.lax.fori_loop(0, T, step, jnp.int32(0))


@jax.jit
def _forward(w_all, b_all, x):
    B, T, C, H, W = x.shape
    f = _FACTOR
    F2 = f * f
    SD = _SD
    L, _, FILT, _ = w_all.shape
    HWp = H * _WP
    C3 = 3 * F2

    perm = (list(range(3 * C, 3 * C + C3 + SD)) + list(range(3 * C))
            + list(range(3 * C + C3 + SD, FILT)))
    w_perm = w_all.at[0].set(w_all[0][..., jnp.array(perm)])
    w_r = w_perm.reshape(L, 3, 3, FILT, FILT)
    w_stack = jnp.transpose(w_r, (0, 2, 3, 1, 4)).reshape(L, 3 * FILT, 3 * FILT)
    b_rep = jnp.broadcast_to(b_all[:, :, None], (L, FILT, 128)).astype(jnp.float32)

    kernel_fn = functools.partial(_step_kernel, W=W, T=T, L=L, F2=F2, SD=SD,
                                  FILT=FILT)

    out_flat = pl.pallas_call(
        kernel_fn,
        out_shape=jax.ShapeDtypeStruct((B, T, 3 * F2, HWp), jnp.bfloat16),
        grid=(B,),
        in_specs=[
            pl.BlockSpec((None, T, C, H, W), lambda b: (b, 0, 0, 0, 0)),
            pl.BlockSpec((L, 3 * FILT, 3 * FILT), lambda b: (0, 0, 0)),
            pl.BlockSpec((L, FILT, 128), lambda b: (0, 0, 0)),
        ],
        out_specs=pl.BlockSpec((None, T, 3 * F2, HWp), lambda b: (b, 0, 0, 0)),
        scratch_shapes=[pltpu.VMEM((T, C, HWp), jnp.bfloat16),
                        pltpu.VMEM((3 * FILT, HWp + 2 * _WP), jnp.bfloat16),
                        pltpu.VMEM((3 * FILT, HWp + 2 * _WP), jnp.bfloat16),
                        pltpu.VMEM((3 * FILT, HWp + 2 * _WP), jnp.bfloat16)],
        compiler_params=pltpu.CompilerParams(
            dimension_semantics=("parallel",)),
    )(x, w_stack, b_rep)

    y = out_flat.reshape(B, T, f, f, C, H, _WP)[:, :, :, :, :, :, 1:1 + W]
    y = jnp.transpose(y, (0, 1, 4, 5, 2, 6, 3))
    return y.reshape(B, T, C, f * H, f * W).astype(jnp.float32)


def kernel(w_all, b_all, x):
    return _forward(w_all, b_all, x)


# speedup vs baseline: 1.0718x; 1.0718x over previous
import functools

import jax
import jax.numpy as jnp
from jax.experimental import pallas as pl
from jax.experimental.pallas import tpu as pltpu

_FACTOR = 2
_SD = 8
_WP = 128
_CH = 512


def _step_kernel(x_ref, w_ref, b_ref, out_ref, xs_ref, xa_ref, xb_ref,
                 xc_ref, *, W, T, L, F2, SD, FILT):
    C = x_ref.shape[1]
    H = x_ref.shape[2]
    HWp = H * _WP
    GP = _WP
    C3 = 3 * F2
    CH = _CH if (HWp % _CH == 0 and HWp >= _CH) else HWp
    NCH = HWp // CH
    F = FILT
    KR = 3 * F + 8

    zc1 = jnp.zeros((C, H, 1), jnp.float32)
    zc2 = jnp.zeros((C, H, _WP - W - 1), jnp.float32)
    for t in range(T):
        xp = jnp.concatenate([zc1, x_ref[t], zc2], axis=2)
        xs_ref[t] = xp.reshape(C, HWp)
    for ref in (xa_ref, xb_ref, xc_ref):
        ref[...] = jnp.zeros(ref.shape, jnp.bfloat16)
        ref[3 * F:3 * F + 1, :] = jnp.ones((1, ref.shape[1]), jnp.bfloat16)

    colv = jax.lax.broadcasted_iota(jnp.int32, (1, CH), 1) % _WP
    real = (colv >= 1) & (colv <= W)

    BLOCKS = ((0, _WP), (F, 0), (2 * F, -_WP))

    def put3(dst_ref, y16, base):
        for blk, off in BLOCKS:
            dst_ref[blk:blk + F, GP + base + off:GP + base + off + CH] = y16

    def conv_chunk(layer, src_ref, base, relu):
        z = jnp.dot(w_ref[layer], src_ref[:KR, GP + base:GP + base + CH],
                    preferred_element_type=jnp.float32)
        zl = pltpu.roll(z[:F], 1, axis=1)
        zr = pltpu.roll(z[2 * F:], CH - 1, axis=1)
        y = z[F:2 * F] + zl + zr
        if relu:
            y = jnp.maximum(y, 0.0)
        return jnp.where(real, y, 0.0)

    def step(t, carry):
        tp = jnp.maximum(t - 1, 0)
        tn = jnp.minimum(t + 1, T - 1)
        fr = jnp.concatenate(
            [xs_ref[tp].astype(jnp.bfloat16),
             xs_ref[t].astype(jnp.bfloat16),
             xs_ref[tn].astype(jnp.bfloat16)], axis=0)
        for blk, off in BLOCKS:
            xa_ref[blk + C3 + SD:blk + C3 + SD + 3 * C,
                   GP + off:GP + off + HWp] = fr

        for j in range(NCH):
            base = j * CH
            y = conv_chunk(0, xa_ref, base, relu=True)
            put3(xb_ref, y.astype(jnp.bfloat16), base)
        for j in range(NCH):
            base = j * CH
            y = conv_chunk(1, xb_ref, base, relu=True)
            put3(xc_ref, y.astype(jnp.bfloat16), base)
        for j in range(NCH):
            base = j * CH
            y = conv_chunk(2, xc_ref, base, relu=False)
            fch = xs_ref[t, :, base:base + CH]
            rgb16 = (y[:C3]
                     + jnp.concatenate([fch] * F2, axis=0)
                     ).astype(jnp.bfloat16)
            st16 = jnp.maximum(y[C3:C3 + SD], 0.0).astype(jnp.bfloat16)
            out_ref[t, :, base:base + CH] = rgb16
            for blk, off in BLOCKS:
                lo = GP + base + off
                xa_ref[blk:blk + C3, lo:lo + CH] = rgb16
                xa_ref[blk + C3:blk + C3 + SD, lo:lo + CH] = st16
        return carry

    jax.lax.fori_loop(0, T, step, jnp.int32(0))


@jax.jit
def _forward(w_all, b_all, x):
    B, T, C, H, W = x.shape
    f = _FACTOR
    F2 = f * f
    SD = _SD
    L, _, FILT, _ = w_all.shape
    HWp = H * _WP
    C3 = 3 * F2

    perm = (list(range(3 * C, 3 * C + C3 + SD)) + list(range(3 * C))
            + list(range(3 * C + C3 + SD, FILT)))
    w_perm = w_all.at[0].set(w_all[0][..., jnp.array(perm)])
    w_r = w_perm.reshape(L, 3, 3, FILT, FILT)
    w_stack = jnp.transpose(w_r, (0, 2, 3, 1, 4)).reshape(L, 3 * FILT, 3 * FILT)
    bcols = jnp.zeros((L, 3 * FILT, 8), jnp.float32)
    bcols = bcols.at[:, FILT:2 * FILT, 0].set(b_all)
    w_stack = jnp.concatenate([w_stack, bcols.astype(w_stack.dtype)], axis=2)
    b_rep = jnp.broadcast_to(b_all[:, :, None], (L, FILT, 128)).astype(jnp.float32)

    kernel_fn = functools.partial(_step_kernel, W=W, T=T, L=L, F2=F2, SD=SD,
                                  FILT=FILT)

    out_flat = pl.pallas_call(
        kernel_fn,
        out_shape=jax.ShapeDtypeStruct((B, T, 3 * F2, HWp), jnp.bfloat16),
        grid=(B,),
        in_specs=[
            pl.BlockSpec((None, T, C, H, W), lambda b: (b, 0, 0, 0, 0)),
            pl.BlockSpec((L, 3 * FILT, 3 * FILT + 8), lambda b: (0, 0, 0)),
            pl.BlockSpec((L, FILT, 128), lambda b: (0, 0, 0)),
        ],
        out_specs=pl.BlockSpec((None, T, 3 * F2, HWp), lambda b: (b, 0, 0, 0)),
        scratch_shapes=[pltpu.VMEM((T, C, HWp), jnp.float32),
                        pltpu.VMEM((3 * FILT + 8, HWp + 2 * _WP), jnp.bfloat16),
                        pltpu.VMEM((3 * FILT + 8, HWp + 2 * _WP), jnp.bfloat16),
                        pltpu.VMEM((3 * FILT + 8, HWp + 2 * _WP), jnp.bfloat16)],
        compiler_params=pltpu.CompilerParams(
            dimension_semantics=("parallel",)),
    )(x, w_stack, b_rep)

    y = out_flat.reshape(B, T, f, f, C, H, _WP)[:, :, :, :, :, :, 1:1 + W]
    y = jnp.transpose(y, (0, 1, 4, 5, 2, 6, 3))
    return y.reshape(B, T, C, f * H, f * W).astype(jnp.float32)


def kernel(w_all, b_all, x):
    return _forward(w_all, b_all, x)


# speedup vs baseline: 1.0768x; 1.0047x over previous
import functools

import jax
import jax.numpy as jnp
from jax.experimental import pallas as pl
from jax.experimental.pallas import tpu as pltpu

_FACTOR = 2
_SD = 8
_WP = 128
_CH = 256


def _step_kernel(x_ref, w_ref, b_ref, out_ref, xs_ref, xa_ref, xb_ref,
                 xc_ref, *, W, T, L, F2, SD, FILT):
    C = x_ref.shape[1]
    H = x_ref.shape[2]
    HWp = H * _WP
    GP = _WP
    C3 = 3 * F2
    CH = _CH if (HWp % _CH == 0 and HWp >= _CH) else HWp
    NCH = HWp // CH
    F = FILT
    KR = 3 * F + 8

    zc1 = jnp.zeros((C, H, 1), jnp.float32)
    zc2 = jnp.zeros((C, H, _WP - W - 1), jnp.float32)
    for t in range(T):
        xp = jnp.concatenate([zc1, x_ref[t], zc2], axis=2)
        xs_ref[t] = xp.reshape(C, HWp)
    for ref in (xa_ref, xb_ref, xc_ref):
        ref[...] = jnp.zeros(ref.shape, jnp.bfloat16)
        ref[3 * F:3 * F + 1, :] = jnp.ones((1, ref.shape[1]), jnp.bfloat16)

    colv = jax.lax.broadcasted_iota(jnp.int32, (1, CH), 1) % _WP
    real = (colv >= 1) & (colv <= W)

    BLOCKS = ((0, _WP), (F, 0), (2 * F, -_WP))

    def put3(dst_ref, y16, base):
        for blk, off in BLOCKS:
            dst_ref[blk:blk + F, GP + base + off:GP + base + off + CH] = y16

    def conv_chunk(layer, src_ref, base, relu):
        z = jnp.dot(w_ref[layer], src_ref[:KR, GP + base:GP + base + CH],
                    preferred_element_type=jnp.float32)
        zl = pltpu.roll(z[:F], 1, axis=1)
        zr = pltpu.roll(z[2 * F:], CH - 1, axis=1)
        y = z[F:2 * F] + zl + zr
        if relu:
            y = jnp.maximum(y, 0.0)
        return jnp.where(real, y, 0.0)

    def step(t, carry):
        tp = jnp.maximum(t - 1, 0)
        tn = jnp.minimum(t + 1, T - 1)
        fr = jnp.concatenate(
            [xs_ref[tp].astype(jnp.bfloat16),
             xs_ref[t].astype(jnp.bfloat16),
             xs_ref[tn].astype(jnp.bfloat16)], axis=0)
        for blk, off in BLOCKS:
            xa_ref[blk + C3 + SD:blk + C3 + SD + 3 * C,
                   GP + off:GP + off + HWp] = fr

        for j in range(NCH):
            base = j * CH
            y = conv_chunk(0, xa_ref, base, relu=True)
            put3(xb_ref, y.astype(jnp.bfloat16), base)
        for j in range(NCH):
            base = j * CH
            y = conv_chunk(1, xb_ref, base, relu=True)
            put3(xc_ref, y.astype(jnp.bfloat16), base)
        for j in range(NCH):
            base = j * CH
            y = conv_chunk(2, xc_ref, base, relu=False)
            fch = xs_ref[t, :, base:base + CH]
            rgb16 = (y[:C3]
                     + jnp.concatenate([fch] * F2, axis=0)
                     ).astype(jnp.bfloat16)
            st16 = jnp.maximum(y[C3:C3 + SD], 0.0).astype(jnp.bfloat16)
            out_ref[t, :, base:base + CH] = rgb16
            for blk, off in BLOCKS:
                lo = GP + base + off
                xa_ref[blk:blk + C3, lo:lo + CH] = rgb16
                xa_ref[blk + C3:blk + C3 + SD, lo:lo + CH] = st16
        return carry

    jax.lax.fori_loop(0, T, step, jnp.int32(0))


@jax.jit
def _forward(w_all, b_all, x):
    B, T, C, H, W = x.shape
    f = _FACTOR
    F2 = f * f
    SD = _SD
    L, _, FILT, _ = w_all.shape
    HWp = H * _WP
    C3 = 3 * F2

    perm = (list(range(3 * C, 3 * C + C3 + SD)) + list(range(3 * C))
            + list(range(3 * C + C3 + SD, FILT)))
    w_perm = w_all.at[0].set(w_all[0][..., jnp.array(perm)])
    w_r = w_perm.reshape(L, 3, 3, FILT, FILT)
    w_stack = jnp.transpose(w_r, (0, 2, 3, 1, 4)).reshape(L, 3 * FILT, 3 * FILT)
    bcols = jnp.zeros((L, 3 * FILT, 8), jnp.float32)
    bcols = bcols.at[:, FILT:2 * FILT, 0].set(b_all)
    w_stack = jnp.concatenate([w_stack, bcols.astype(w_stack.dtype)], axis=2)
    b_rep = jnp.broadcast_to(b_all[:, :, None], (L, FILT, 128)).astype(jnp.float32)

    kernel_fn = functools.partial(_step_kernel, W=W, T=T, L=L, F2=F2, SD=SD,
                                  FILT=FILT)

    out_flat = pl.pallas_call(
        kernel_fn,
        out_shape=jax.ShapeDtypeStruct((B, T, 3 * F2, HWp), jnp.bfloat16),
        grid=(B,),
        in_specs=[
            pl.BlockSpec((None, T, C, H, W), lambda b: (b, 0, 0, 0, 0)),
            pl.BlockSpec((L, 3 * FILT, 3 * FILT + 8), lambda b: (0, 0, 0)),
            pl.BlockSpec((L, FILT, 128), lambda b: (0, 0, 0)),
        ],
        out_specs=pl.BlockSpec((None, T, 3 * F2, HWp), lambda b: (b, 0, 0, 0)),
        scratch_shapes=[pltpu.VMEM((T, C, HWp), jnp.float32),
                        pltpu.VMEM((3 * FILT + 8, HWp + 2 * _WP), jnp.bfloat16),
                        pltpu.VMEM((3 * FILT + 8, HWp + 2 * _WP), jnp.bfloat16),
                        pltpu.VMEM((3 * FILT + 8, HWp + 2 * _WP), jnp.bfloat16)],
        compiler_params=pltpu.CompilerParams(
            dimension_semantics=("parallel",)),
    )(x, w_stack, b_rep)

    y = out_flat.reshape(B, T, f, f, C, H, _WP)[:, :, :, :, :, :, 1:1 + W]
    y = jnp.transpose(y, (0, 1, 4, 5, 2, 6, 3))
    return y.reshape(B, T, C, f * H, f * W).astype(jnp.float32)


def kernel(w_all, b_all, x):
    return _forward(w_all, b_all, x)
